# baseline (speedup 1.0000x reference)
import functools

import jax
import jax.numpy as jnp
from jax import lax
from jax.experimental import pallas as pl
from jax.experimental.pallas import tpu as pltpu
from jax.experimental.pallas import tpu_sc as plsc

_LANES = 16
_CHUNK = 256


def _mlp_body(x_ref, w1_ref, b1_ref, w2_ref, b2_ref, logits_ref):
    h = jnp.tanh(
        jnp.dot(x_ref[...], w1_ref[...], preferred_element_type=jnp.float32)
        + b1_ref[...]
    )
    logits_ref[...] = (
        jnp.dot(h, w2_ref[...], preferred_element_type=jnp.float32) + b2_ref[...]
    )


def _tc_logits(x, W1, b1, W2, b2):
    n, d = x.shape
    h_dim = W1.shape[1]
    e = W2.shape[1]
    bm = 4096
    return pl.pallas_call(
        _mlp_body,
        grid=(n // bm,),
        in_specs=[
            pl.BlockSpec((bm, d), lambda i: (i, 0)),
            pl.BlockSpec((d, h_dim), lambda i: (0, 0)),
            pl.BlockSpec((1, h_dim), lambda i: (0, 0)),
            pl.BlockSpec((h_dim, e), lambda i: (0, 0)),
            pl.BlockSpec((1, e), lambda i: (0, 0)),
        ],
        out_specs=pl.BlockSpec((bm, e), lambda i: (i, 0)),
        out_shape=jax.ShapeDtypeStruct((n, e), jnp.float32),
    )(x, W1, b1.reshape(1, -1), W2, b2.reshape(1, -1))


def _sc_gates_kernel(n_tokens, num_experts):
    info = plsc.get_sparse_core_info()
    nc, ns = info.num_cores, info.num_subcores
    n_workers = nc * ns
    per_worker = n_tokens // n_workers
    n_chunks = per_worker // _CHUNK
    groups_per_chunk = _CHUNK // _LANES
    chunk_elems = _CHUNK * num_experts
    mesh = plsc.VectorSubcoreMesh(core_axis_name="c", subcore_axis_name="s")

    @functools.partial(
        pl.kernel,
        out_type=jax.ShapeDtypeStruct((n_tokens * num_experts,), jnp.float32),
        mesh=mesh,
        scratch_types=[
            pltpu.VMEM((chunk_elems,), jnp.float32),
            pltpu.VMEM((chunk_elems,), jnp.float32),
        ],
        compiler_params=pltpu.CompilerParams(needs_layout_passes=False),
    )
    def gates_kernel(logits_hbm, gates_hbm, lbuf, gbuf):
        wid = lax.axis_index("s") * nc + lax.axis_index("c")
        lane = lax.iota(jnp.int32, _LANES)
        zero = jnp.zeros((_LANES,), jnp.float32)

        def chunk_body(c, _):
            base = (wid * per_worker + c * _CHUNK) * num_experts
            pltpu.sync_copy(logits_hbm.at[pl.ds(base, chunk_elems)], lbuf)

            def group_body(g, _):
                row0 = (g * _LANES + lane) * num_experts
                m1 = jnp.full((_LANES,), -jnp.inf, jnp.float32)
                m2 = jnp.full((_LANES,), -jnp.inf, jnp.float32)
                i1 = jnp.zeros((_LANES,), jnp.int32)
                i2 = jnp.zeros((_LANES,), jnp.int32)
                for e in range(num_experts):
                    col = jnp.full((_LANES,), e, jnp.int32)
                    v = plsc.load_gather(lbuf, [row0 + e])
                    gt1 = v > m1
                    was2 = jnp.logical_and(jnp.logical_not(gt1), v > m2)
                    i2 = jnp.where(gt1, i1, jnp.where(was2, col, i2))
                    m2 = jnp.where(gt1, m1, jnp.where(was2, v, m2))
                    i1 = jnp.where(gt1, col, i1)
                    m1 = jnp.where(gt1, v, m1)
                    plsc.store_scatter(gbuf, [row0 + e], zero)
                e2 = jnp.exp(m2 - m1)
                g1 = 1.0 / (1.0 + e2)
                g2 = 1.0 - g1
                plsc.store_scatter(gbuf, [row0 + i1], g1)
                plsc.store_scatter(gbuf, [row0 + i2], g2)
                return 0

            lax.fori_loop(0, groups_per_chunk, group_body, 0)
            pltpu.sync_copy(gbuf, gates_hbm.at[pl.ds(base, chunk_elems)])
            return 0

        lax.fori_loop(0, n_chunks, chunk_body, 0)

    return gates_kernel


def kernel(x, W1, b1, W2, b2):
    n = x.shape[0]
    e = W2.shape[1]
    logits = _tc_logits(x, W1, b1, W2, b2)
    gates_flat = _sc_gates_kernel(n, e)(logits.reshape(-1))
    return (gates_flat.reshape(n, e), logits)
